# baseline (speedup 1.0000x reference)
import functools

import jax
import jax.numpy as jnp
from jax import lax
from jax.experimental import pallas as pl
from jax.experimental.pallas import tpu as pltpu
from jax.experimental.pallas import tpu_sc as plsc

N = 10000
E = 320000
D = 128
H = 128
FF = 256

NC = 2
NS = 16
CHUNK = 40
EDGES_PER_TILE = E // (NC * NS)
NCHUNKS = EDGES_PER_TILE // CHUNK
ONES_PAD = ((CHUNK + 15) // 16) * 16
NPAD = 10112
ROWS_PER_TILE = NPAD // NS
DEG_PAD = 10240
DEG_TILE = DEG_PAD // NS
RRING = 8
GDEPTH = RRING - 1
IRING = RRING + 2

ROW_BLOCK = 1024
GRID = 10

_MESH = plsc.VectorSubcoreMesh(
    core_axis_name="c", subcore_axis_name="s", num_cores=NC, num_subcores=NS)


@functools.partial(
    pl.kernel,
    mesh=_MESH,
    out_type=jax.ShapeDtypeStruct((NC, DEG_PAD), jnp.float32),
    scratch_types=[
        pltpu.VMEM((NCHUNKS, CHUNK), jnp.int32),
        pltpu.VMEM((ONES_PAD,), jnp.float32),
        pltpu.VMEM((DEG_TILE,), jnp.float32),
        pltpu.SemaphoreType.DMA,
        pltpu.VMEM_SHARED((DEG_PAD,), jnp.float32),
    ],
)
def _sc_degree(ei_hbm, out_hbm, dstv, ones_v, zv, ssem, acc_sh):
    cid = lax.axis_index("c")
    sid = lax.axis_index("s")
    wid = cid * NS + sid

    def ones_body(i, c):
        ones_v[pl.ds(i * 16, 16)] = jnp.ones((16,), jnp.float32)
        return c
    lax.fori_loop(0, ONES_PAD // 16, ones_body, 0)

    def zeros_body(i, c):
        zv[pl.ds(i * 16, 16)] = jnp.zeros((16,), jnp.float32)
        return c
    lax.fori_loop(0, DEG_TILE // 16, zeros_body, 0)

    pltpu.sync_copy(ei_hbm.at[1, wid], dstv)
    pltpu.sync_copy(zv, acc_sh.at[pl.ds(sid * DEG_TILE, DEG_TILE)])
    plsc.subcore_barrier()

    def body(i, c):
        pltpu.async_copy(ones_v.at[pl.ds(0, CHUNK)], acc_sh.at[dstv.at[i]], ssem, add=True)
        return c
    lax.fori_loop(0, NCHUNKS, body, 0)

    def drain(i, c):
        pltpu.make_async_copy(ones_v.at[pl.ds(0, CHUNK)], acc_sh.at[dstv.at[i]], ssem).wait()
        return c
    lax.fori_loop(0, NCHUNKS, drain, 0)

    plsc.subcore_barrier()
    pltpu.sync_copy(acc_sh.at[pl.ds(sid * DEG_TILE, DEG_TILE)],
                    out_hbm.at[cid, pl.ds(sid * DEG_TILE, DEG_TILE)])


@functools.partial(
    pl.kernel,
    mesh=_MESH,
    out_type=jax.ShapeDtypeStruct((NC, NPAD, H), jnp.float32),
    scratch_types=[
        pltpu.VMEM((IRING, CHUNK), jnp.int32),
        pltpu.VMEM((IRING, CHUNK), jnp.int32),
        pltpu.VMEM((RRING, CHUNK, H), jnp.float32),
        pltpu.SemaphoreType.DMA,
        pltpu.SemaphoreType.DMA,
        pltpu.SemaphoreType.DMA,
        pltpu.VMEM_SHARED((NPAD, H), jnp.float32),
    ],
)
def _sc_scatter(hs_hbm, ei_hbm, z_hbm, out_hbm, srcv, dstv, rows_v,
                isem, gsem, ssem, acc_sh):
    cid = lax.axis_index("c")
    sid = lax.axis_index("s")
    wid = cid * NS + sid

    pltpu.sync_copy(z_hbm, acc_sh.at[pl.ds(sid * ROWS_PER_TILE, ROWS_PER_TILE)])

    def _load_idx(j, slot, sync):
        if sync:
            pltpu.sync_copy(ei_hbm.at[0, wid, j], srcv.at[slot])
            pltpu.sync_copy(ei_hbm.at[1, wid, j], dstv.at[slot])
        else:
            pltpu.async_copy(ei_hbm.at[0, wid, j], srcv.at[slot], isem)
            pltpu.async_copy(ei_hbm.at[1, wid, j], dstv.at[slot], isem)

    for j in range(GDEPTH):
        _load_idx(j, j, True)
        pltpu.async_copy(hs_hbm.at[srcv.at[j]], rows_v.at[j], gsem)
    _load_idx(GDEPTH, GDEPTH, False)
    plsc.subcore_barrier()

    def body(i, c):
        b = lax.rem(i, RRING)
        pltpu.make_async_copy(hs_hbm.at[srcv.at[lax.rem(i, IRING)]],
                              rows_v.at[b], gsem).wait()
        pltpu.async_copy(rows_v.at[b], acc_sh.at[dstv.at[lax.rem(i, IRING)]],
                         ssem, add=True)

        @pl.when(i + GDEPTH < NCHUNKS)
        def _():
            m = lax.rem(i + GDEPTH, IRING)
            pltpu.make_async_copy(ei_hbm.at[0, wid, i + GDEPTH], srcv.at[m],
                                  isem).wait()
            pltpu.make_async_copy(ei_hbm.at[1, wid, i + GDEPTH], dstv.at[m],
                                  isem).wait()
            r = lax.rem(i + GDEPTH, RRING)

            @pl.when(i >= 1)
            def _():
                pltpu.make_async_copy(
                    rows_v.at[r], acc_sh.at[dstv.at[lax.rem(i - 1, IRING)]],
                    ssem).wait()
            pltpu.async_copy(hs_hbm.at[srcv.at[m]], rows_v.at[r], gsem)

        @pl.when(i + GDEPTH + 1 < NCHUNKS)
        def _():
            _load_idx(i + GDEPTH + 1, lax.rem(i + GDEPTH + 1, IRING), False)
        return c
    lax.fori_loop(0, NCHUNKS, body, 0)

    def sdrain(j, c):
        pltpu.make_async_copy(rows_v.at[lax.rem(j, RRING)],
                              acc_sh.at[dstv.at[lax.rem(j, IRING)]],
                              ssem).wait()
        return c
    lax.fori_loop(NCHUNKS - RRING, NCHUNKS, sdrain, 0)

    plsc.subcore_barrier()
    pltpu.sync_copy(acc_sh.at[pl.ds(sid * ROWS_PER_TILE, ROWS_PER_TILE)],
                    out_hbm.at[cid, pl.ds(sid * ROWS_PER_TILE, ROWS_PER_TILE)])


def _dinv_block(degp_blk):
    return lax.rsqrt(degp_blk[0] + degp_blk[1] + 1.0)[:, None]


def _tc1_body(x_ref, w_ref, degp_ref, hs_ref):
    h = jnp.dot(x_ref[...], w_ref[...], preferred_element_type=jnp.float32)
    hs_ref[...] = h * _dinv_block(degp_ref[...])


def _tc_scale(x, W_gcn, degp):
    return pl.pallas_call(
        _tc1_body,
        grid=(GRID,),
        in_specs=[
            pl.BlockSpec((ROW_BLOCK, D), lambda i: (i, 0)),
            pl.BlockSpec((D, H), lambda i: (0, 0)),
            pl.BlockSpec((2, ROW_BLOCK), lambda i: (0, i)),
        ],
        out_specs=pl.BlockSpec((ROW_BLOCK, H), lambda i: (i, 0)),
        out_shape=jax.ShapeDtypeStruct((N, H), jnp.float32),
    )(x, W_gcn, degp)


def _ln(v, gamma, beta, eps=1e-5):
    mu = jnp.mean(v, axis=-1, keepdims=True)
    var = jnp.mean((v - mu) * (v - mu), axis=-1, keepdims=True)
    return (v - mu) * lax.rsqrt(var + eps) * gamma + beta


def _tc2_body(sp_ref, hs_ref, degp_ref, x_ref, bg_ref, w1_ref, b1_ref, w2_ref,
              b2_ref, g1_ref, be1_ref, g2_ref, be2_ref, out_ref):
    s = sp_ref[0] + sp_ref[1]
    agg = _dinv_block(degp_ref[...]) * (s + hs_ref[...]) + bg_ref[...]
    xr = x_ref[...] + agg
    xn = _ln(xr, g1_ref[...], be1_ref[...])
    t = jnp.maximum(
        jnp.dot(xn, w1_ref[...], preferred_element_type=jnp.float32) + b1_ref[...],
        0.0)
    ff = jnp.dot(t, w2_ref[...], preferred_element_type=jnp.float32) + b2_ref[...]
    out_ref[...] = _ln(xn + ff, g2_ref[...], be2_ref[...])


def _tc_epilogue(sp, hs, degp, x, b_gcn, W1, b1, W2, b2, g1, be1, g2, be2):
    full = lambda shape: pl.BlockSpec(shape, lambda i: tuple(0 for _ in shape))
    return pl.pallas_call(
        _tc2_body,
        grid=(GRID,),
        in_specs=[
            pl.BlockSpec((NC, ROW_BLOCK, H), lambda i: (0, i, 0)),
            pl.BlockSpec((ROW_BLOCK, H), lambda i: (i, 0)),
            pl.BlockSpec((2, ROW_BLOCK), lambda i: (0, i)),
            pl.BlockSpec((ROW_BLOCK, D), lambda i: (i, 0)),
            full((H,)),
            full((H, FF)),
            full((FF,)),
            full((FF, H)),
            full((H,)),
            full((H,)),
            full((H,)),
            full((H,)),
            full((H,)),
        ],
        out_specs=pl.BlockSpec((ROW_BLOCK, H), lambda i: (i, 0)),
        out_shape=jax.ShapeDtypeStruct((N, H), jnp.float32),
    )(sp, hs, degp, x, b_gcn, W1, b1, W2, b2, g1, be1, g2, be2)


def kernel(x, edge_index, W_gcn, b_gcn, W1, b1, W2, b2, g1, be1, g2, be2):
    ei4 = edge_index.astype(jnp.int32).reshape(2, NC * NS, NCHUNKS, CHUNK)

    degp = _sc_degree(ei4)
    hs = _tc_scale(x, W_gcn, degp)
    sp = _sc_scatter(hs, ei4,
                     jnp.zeros((ROWS_PER_TILE, H), jnp.float32))
    return _tc_epilogue(sp, hs, degp, x, b_gcn, W1, b1, W2, b2,
                        g1, be1, g2, be2)


# speedup vs baseline: 1.3686x; 1.3686x over previous
import functools

import jax
import jax.numpy as jnp
from jax import lax
from jax.experimental import pallas as pl
from jax.experimental.pallas import tpu as pltpu
from jax.experimental.pallas import tpu_sc as plsc

N = 10000
E = 320000
D = 128
H = 128
FF = 256

NC = 2
NS = 16
CHUNK = 80
EDGES_PER_TILE = E // (NC * NS)
NCHUNKS = EDGES_PER_TILE // CHUNK
ONES_PAD = ((CHUNK + 15) // 16) * 16
NPAD = 10112
ROWS_PER_TILE = NPAD // NS
DEG_PAD = 10240
DEG_TILE = DEG_PAD // NS
RRING = 4
GDEPTH = RRING - 1
IRING = 8

ROW_BLOCK = 1024
GRID = 10

_MESH = plsc.VectorSubcoreMesh(
    core_axis_name="c", subcore_axis_name="s", num_cores=NC, num_subcores=NS)


@functools.partial(
    pl.kernel,
    mesh=_MESH,
    out_type=jax.ShapeDtypeStruct((NC, DEG_PAD), jnp.float32),
    scratch_types=[
        pltpu.VMEM((NCHUNKS, 2, CHUNK), jnp.int32),
        pltpu.VMEM((ONES_PAD,), jnp.float32),
        pltpu.VMEM((DEG_TILE,), jnp.float32),
        pltpu.SemaphoreType.DMA,
        pltpu.VMEM_SHARED((DEG_PAD,), jnp.float32),
    ],
)
def _sc_degree(ei_hbm, out_hbm, dstv, ones_v, zv, ssem, acc_sh):
    cid = lax.axis_index("c")
    sid = lax.axis_index("s")
    wid = cid * NS + sid

    def ones_body(i, c):
        ones_v[pl.ds(i * 16, 16)] = jnp.ones((16,), jnp.float32)
        return c
    lax.fori_loop(0, ONES_PAD // 16, ones_body, 0)

    def zeros_body(i, c):
        zv[pl.ds(i * 16, 16)] = jnp.zeros((16,), jnp.float32)
        return c
    lax.fori_loop(0, DEG_TILE // 16, zeros_body, 0)

    pltpu.sync_copy(ei_hbm.at[wid], dstv)
    pltpu.sync_copy(zv, acc_sh.at[pl.ds(sid * DEG_TILE, DEG_TILE)])
    plsc.subcore_barrier()

    def body(i, c):
        pltpu.async_copy(ones_v.at[pl.ds(0, CHUNK)], acc_sh.at[dstv.at[i, 1]], ssem, add=True)
        return c
    lax.fori_loop(0, NCHUNKS, body, 0)

    def drain(i, c):
        pltpu.make_async_copy(ones_v.at[pl.ds(0, CHUNK)], acc_sh.at[dstv.at[i, 1]], ssem).wait()
        return c
    lax.fori_loop(0, NCHUNKS, drain, 0)

    plsc.subcore_barrier()
    pltpu.sync_copy(acc_sh.at[pl.ds(sid * DEG_TILE, DEG_TILE)],
                    out_hbm.at[cid, pl.ds(sid * DEG_TILE, DEG_TILE)])


@functools.partial(
    pl.kernel,
    mesh=_MESH,
    out_type=jax.ShapeDtypeStruct((NC, NPAD, H), jnp.float32),
    scratch_types=[
        pltpu.VMEM((IRING, 2, CHUNK), jnp.int32),
        pltpu.VMEM((RRING, CHUNK, H), jnp.float32),
        pltpu.SemaphoreType.DMA,
        pltpu.SemaphoreType.DMA,
        pltpu.SemaphoreType.DMA,
        pltpu.VMEM_SHARED((NPAD, H), jnp.float32),
    ],
)
def _sc_scatter(hs_hbm, ei_hbm, z_hbm, out_hbm, idxv, rows_v,
                isem, gsem, ssem, acc_sh):
    cid = lax.axis_index("c")
    sid = lax.axis_index("s")
    wid = cid * NS + sid

    pltpu.sync_copy(z_hbm, acc_sh.at[pl.ds(sid * ROWS_PER_TILE, ROWS_PER_TILE)])

    def _load_idx(j, slot, sync):
        if sync:
            pltpu.sync_copy(ei_hbm.at[wid, j], idxv.at[slot])
        else:
            pltpu.async_copy(ei_hbm.at[wid, j], idxv.at[slot], isem)

    for j in range(GDEPTH):
        _load_idx(j, j, True)
        pltpu.async_copy(hs_hbm.at[idxv.at[j, 0]], rows_v.at[j], gsem)
    _load_idx(GDEPTH, GDEPTH, False)
    plsc.subcore_barrier()

    def body(i, c):
        b = i & (RRING - 1)
        q = i & (IRING - 1)
        pltpu.make_async_copy(hs_hbm.at[idxv.at[q, 0]],
                              rows_v.at[b], gsem).wait()
        pltpu.async_copy(rows_v.at[b], acc_sh.at[idxv.at[q, 1]],
                         ssem, add=True)

        @pl.when(i + GDEPTH < NCHUNKS)
        def _():
            m = (i + GDEPTH) & (IRING - 1)
            pltpu.make_async_copy(ei_hbm.at[wid, i + GDEPTH], idxv.at[m],
                                  isem).wait()
            r = (i + GDEPTH) & (RRING - 1)

            @pl.when(i >= 1)
            def _():
                pltpu.make_async_copy(
                    rows_v.at[r], acc_sh.at[idxv.at[(i - 1) & (IRING - 1), 1]],
                    ssem).wait()
            pltpu.async_copy(hs_hbm.at[idxv.at[m, 0]], rows_v.at[r], gsem)

        @pl.when(i + GDEPTH + 1 < NCHUNKS)
        def _():
            _load_idx(i + GDEPTH + 1, (i + GDEPTH + 1) & (IRING - 1), False)
        return c
    lax.fori_loop(0, NCHUNKS, body, 0)

    def sdrain(j, c):
        pltpu.make_async_copy(rows_v.at[j & (RRING - 1)],
                              acc_sh.at[idxv.at[j & (IRING - 1), 1]],
                              ssem).wait()
        return c
    lax.fori_loop(NCHUNKS - RRING, NCHUNKS, sdrain, 0)

    plsc.subcore_barrier()
    pltpu.sync_copy(acc_sh.at[pl.ds(sid * ROWS_PER_TILE, ROWS_PER_TILE)],
                    out_hbm.at[cid, pl.ds(sid * ROWS_PER_TILE, ROWS_PER_TILE)])


def _dinv_block(degp_blk):
    return lax.rsqrt(degp_blk[0] + degp_blk[1] + 1.0)[:, None]


def _tc1_body(x_ref, w_ref, degp_ref, hs_ref):
    h = jnp.dot(x_ref[...], w_ref[...], preferred_element_type=jnp.float32)
    hs_ref[...] = h * _dinv_block(degp_ref[...])


def _tc_scale(x, W_gcn, degp):
    return pl.pallas_call(
        _tc1_body,
        grid=(GRID,),
        in_specs=[
            pl.BlockSpec((ROW_BLOCK, D), lambda i: (i, 0)),
            pl.BlockSpec((D, H), lambda i: (0, 0)),
            pl.BlockSpec((2, ROW_BLOCK), lambda i: (0, i)),
        ],
        out_specs=pl.BlockSpec((ROW_BLOCK, H), lambda i: (i, 0)),
        out_shape=jax.ShapeDtypeStruct((N, H), jnp.float32),
    )(x, W_gcn, degp)


def _ln(v, gamma, beta, eps=1e-5):
    mu = jnp.mean(v, axis=-1, keepdims=True)
    var = jnp.mean((v - mu) * (v - mu), axis=-1, keepdims=True)
    return (v - mu) * lax.rsqrt(var + eps) * gamma + beta


def _tc2_body(sp_ref, hs_ref, degp_ref, x_ref, bg_ref, w1_ref, b1_ref, w2_ref,
              b2_ref, g1_ref, be1_ref, g2_ref, be2_ref, out_ref):
    s = sp_ref[0] + sp_ref[1]
    agg = _dinv_block(degp_ref[...]) * (s + hs_ref[...]) + bg_ref[...]
    xr = x_ref[...] + agg
    xn = _ln(xr, g1_ref[...], be1_ref[...])
    t = jnp.maximum(
        jnp.dot(xn, w1_ref[...], preferred_element_type=jnp.float32) + b1_ref[...],
        0.0)
    ff = jnp.dot(t, w2_ref[...], preferred_element_type=jnp.float32) + b2_ref[...]
    out_ref[...] = _ln(xn + ff, g2_ref[...], be2_ref[...])


def _tc_epilogue(sp, hs, degp, x, b_gcn, W1, b1, W2, b2, g1, be1, g2, be2):
    full = lambda shape: pl.BlockSpec(shape, lambda i: tuple(0 for _ in shape))
    return pl.pallas_call(
        _tc2_body,
        grid=(GRID,),
        in_specs=[
            pl.BlockSpec((NC, ROW_BLOCK, H), lambda i: (0, i, 0)),
            pl.BlockSpec((ROW_BLOCK, H), lambda i: (i, 0)),
            pl.BlockSpec((2, ROW_BLOCK), lambda i: (0, i)),
            pl.BlockSpec((ROW_BLOCK, D), lambda i: (i, 0)),
            full((H,)),
            full((H, FF)),
            full((FF,)),
            full((FF, H)),
            full((H,)),
            full((H,)),
            full((H,)),
            full((H,)),
            full((H,)),
        ],
        out_specs=pl.BlockSpec((ROW_BLOCK, H), lambda i: (i, 0)),
        out_shape=jax.ShapeDtypeStruct((N, H), jnp.float32),
    )(sp, hs, degp, x, b_gcn, W1, b1, W2, b2, g1, be1, g2, be2)


def kernel(x, edge_index, W_gcn, b_gcn, W1, b1, W2, b2, g1, be1, g2, be2):
    ei5 = edge_index.astype(jnp.int32).reshape(
        2, NC * NS, NCHUNKS, CHUNK).transpose(1, 2, 0, 3)

    degp = _sc_degree(ei5)
    hs = _tc_scale(x, W_gcn, degp)
    sp = _sc_scatter(hs, ei5,
                     jnp.zeros((ROWS_PER_TILE, H), jnp.float32))
    return _tc_epilogue(sp, hs, degp, x, b_gcn, W1, b1, W2, b2,
                        g1, be1, g2, be2)


# speedup vs baseline: 1.3977x; 1.0212x over previous
import functools

import jax
import jax.numpy as jnp
from jax import lax
from jax.experimental import pallas as pl
from jax.experimental.pallas import tpu as pltpu
from jax.experimental.pallas import tpu_sc as plsc

N = 10000
E = 320000
D = 128
H = 128
FF = 256

NC = 2
NS = 16
CHUNK = 100
EDGES_PER_TILE = E // (NC * NS)
NCHUNKS = EDGES_PER_TILE // CHUNK
ONES_PAD = ((CHUNK + 15) // 16) * 16
NPAD = 10112
ROWS_PER_TILE = NPAD // NS
DEG_PAD = 10240
DEG_TILE = DEG_PAD // NS
RRING = 3
GDEPTH = RRING - 1
IRING = 8

ROW_BLOCK = 1024
GRID = 10

_MESH = plsc.VectorSubcoreMesh(
    core_axis_name="c", subcore_axis_name="s", num_cores=NC, num_subcores=NS)


@functools.partial(
    pl.kernel,
    mesh=_MESH,
    out_type=jax.ShapeDtypeStruct((NC, DEG_PAD), jnp.float32),
    scratch_types=[
        pltpu.VMEM((NCHUNKS, 2, CHUNK), jnp.int32),
        pltpu.VMEM((ONES_PAD,), jnp.float32),
        pltpu.VMEM((DEG_TILE,), jnp.float32),
        pltpu.SemaphoreType.DMA,
        pltpu.VMEM_SHARED((DEG_PAD,), jnp.float32),
    ],
)
def _sc_degree(ei_hbm, out_hbm, dstv, ones_v, zv, ssem, acc_sh):
    cid = lax.axis_index("c")
    sid = lax.axis_index("s")
    wid = cid * NS + sid

    def ones_body(i, c):
        ones_v[pl.ds(i * 16, 16)] = jnp.ones((16,), jnp.float32)
        return c
    lax.fori_loop(0, ONES_PAD // 16, ones_body, 0)

    def zeros_body(i, c):
        zv[pl.ds(i * 16, 16)] = jnp.zeros((16,), jnp.float32)
        return c
    lax.fori_loop(0, DEG_TILE // 16, zeros_body, 0)

    pltpu.sync_copy(ei_hbm.at[wid], dstv)
    pltpu.sync_copy(zv, acc_sh.at[pl.ds(sid * DEG_TILE, DEG_TILE)])
    plsc.subcore_barrier()

    def body(i, c):
        pltpu.async_copy(ones_v.at[pl.ds(0, CHUNK)], acc_sh.at[dstv.at[i, 1]], ssem, add=True)
        return c
    lax.fori_loop(0, NCHUNKS, body, 0)

    def drain(i, c):
        pltpu.make_async_copy(ones_v.at[pl.ds(0, CHUNK)], acc_sh.at[dstv.at[i, 1]], ssem).wait()
        return c
    lax.fori_loop(0, NCHUNKS, drain, 0)

    plsc.subcore_barrier()
    pltpu.sync_copy(acc_sh.at[pl.ds(sid * DEG_TILE, DEG_TILE)],
                    out_hbm.at[cid, pl.ds(sid * DEG_TILE, DEG_TILE)])


@functools.partial(
    pl.kernel,
    mesh=_MESH,
    out_type=jax.ShapeDtypeStruct((NC, NPAD, H), jnp.float32),
    scratch_types=[
        pltpu.VMEM((IRING, 2, CHUNK), jnp.int32),
        pltpu.VMEM((RRING, CHUNK, H), jnp.float32),
        pltpu.SemaphoreType.DMA,
        pltpu.SemaphoreType.DMA,
        pltpu.SemaphoreType.DMA,
        pltpu.VMEM_SHARED((NPAD, H), jnp.float32),
    ],
)
def _sc_scatter(hs_hbm, ei_hbm, z_hbm, out_hbm, idxv, rows_v,
                isem, gsem, ssem, acc_sh):
    cid = lax.axis_index("c")
    sid = lax.axis_index("s")
    wid = cid * NS + sid

    pltpu.sync_copy(z_hbm, acc_sh.at[pl.ds(sid * ROWS_PER_TILE, ROWS_PER_TILE)])

    def _load_idx(j, slot, sync):
        if sync:
            pltpu.sync_copy(ei_hbm.at[wid, j], idxv.at[slot])
        else:
            pltpu.async_copy(ei_hbm.at[wid, j], idxv.at[slot], isem)

    for j in range(GDEPTH):
        _load_idx(j, j, True)
        pltpu.async_copy(hs_hbm.at[idxv.at[j, 0]], rows_v.at[j], gsem)
    _load_idx(GDEPTH, GDEPTH, False)
    plsc.subcore_barrier()

    def body(i, c):
        b = lax.rem(i, RRING)
        q = i & (IRING - 1)
        pltpu.make_async_copy(hs_hbm.at[idxv.at[q, 0]],
                              rows_v.at[b], gsem).wait()
        pltpu.async_copy(rows_v.at[b], acc_sh.at[idxv.at[q, 1]],
                         ssem, add=True)

        @pl.when(i + GDEPTH < NCHUNKS)
        def _():
            m = (i + GDEPTH) & (IRING - 1)
            pltpu.make_async_copy(ei_hbm.at[wid, i + GDEPTH], idxv.at[m],
                                  isem).wait()
            r = lax.rem(i + GDEPTH, RRING)

            @pl.when(i >= 1)
            def _():
                pltpu.make_async_copy(
                    rows_v.at[r], acc_sh.at[idxv.at[(i - 1) & (IRING - 1), 1]],
                    ssem).wait()
            pltpu.async_copy(hs_hbm.at[idxv.at[m, 0]], rows_v.at[r], gsem)

        @pl.when(i + GDEPTH + 1 < NCHUNKS)
        def _():
            _load_idx(i + GDEPTH + 1, (i + GDEPTH + 1) & (IRING - 1), False)
        return c
    lax.fori_loop(0, NCHUNKS, body, 0)

    def sdrain(j, c):
        pltpu.make_async_copy(rows_v.at[lax.rem(j, RRING)],
                              acc_sh.at[idxv.at[j & (IRING - 1), 1]],
                              ssem).wait()
        return c
    lax.fori_loop(NCHUNKS - RRING, NCHUNKS, sdrain, 0)

    plsc.subcore_barrier()
    pltpu.sync_copy(acc_sh.at[pl.ds(sid * ROWS_PER_TILE, ROWS_PER_TILE)],
                    out_hbm.at[cid, pl.ds(sid * ROWS_PER_TILE, ROWS_PER_TILE)])


def _dinv_block(degp_blk):
    return lax.rsqrt(degp_blk[0] + degp_blk[1] + 1.0)[:, None]


def _tc1_body(x_ref, w_ref, degp_ref, hs_ref):
    h = jnp.dot(x_ref[...], w_ref[...], preferred_element_type=jnp.float32)
    hs_ref[...] = h * _dinv_block(degp_ref[...])


def _tc_scale(x, W_gcn, degp):
    return pl.pallas_call(
        _tc1_body,
        grid=(GRID,),
        in_specs=[
            pl.BlockSpec((ROW_BLOCK, D), lambda i: (i, 0)),
            pl.BlockSpec((D, H), lambda i: (0, 0)),
            pl.BlockSpec((2, ROW_BLOCK), lambda i: (0, i)),
        ],
        out_specs=pl.BlockSpec((ROW_BLOCK, H), lambda i: (i, 0)),
        out_shape=jax.ShapeDtypeStruct((N, H), jnp.float32),
    )(x, W_gcn, degp)


def _ln(v, gamma, beta, eps=1e-5):
    mu = jnp.mean(v, axis=-1, keepdims=True)
    var = jnp.mean((v - mu) * (v - mu), axis=-1, keepdims=True)
    return (v - mu) * lax.rsqrt(var + eps) * gamma + beta


def _tc2_body(sp_ref, hs_ref, degp_ref, x_ref, bg_ref, w1_ref, b1_ref, w2_ref,
              b2_ref, g1_ref, be1_ref, g2_ref, be2_ref, out_ref):
    s = sp_ref[0] + sp_ref[1]
    agg = _dinv_block(degp_ref[...]) * (s + hs_ref[...]) + bg_ref[...]
    xr = x_ref[...] + agg
    xn = _ln(xr, g1_ref[...], be1_ref[...])
    t = jnp.maximum(
        jnp.dot(xn, w1_ref[...], preferred_element_type=jnp.float32) + b1_ref[...],
        0.0)
    ff = jnp.dot(t, w2_ref[...], preferred_element_type=jnp.float32) + b2_ref[...]
    out_ref[...] = _ln(xn + ff, g2_ref[...], be2_ref[...])


def _tc_epilogue(sp, hs, degp, x, b_gcn, W1, b1, W2, b2, g1, be1, g2, be2):
    full = lambda shape: pl.BlockSpec(shape, lambda i: tuple(0 for _ in shape))
    return pl.pallas_call(
        _tc2_body,
        grid=(GRID,),
        in_specs=[
            pl.BlockSpec((NC, ROW_BLOCK, H), lambda i: (0, i, 0)),
            pl.BlockSpec((ROW_BLOCK, H), lambda i: (i, 0)),
            pl.BlockSpec((2, ROW_BLOCK), lambda i: (0, i)),
            pl.BlockSpec((ROW_BLOCK, D), lambda i: (i, 0)),
            full((H,)),
            full((H, FF)),
            full((FF,)),
            full((FF, H)),
            full((H,)),
            full((H,)),
            full((H,)),
            full((H,)),
            full((H,)),
        ],
        out_specs=pl.BlockSpec((ROW_BLOCK, H), lambda i: (i, 0)),
        out_shape=jax.ShapeDtypeStruct((N, H), jnp.float32),
    )(sp, hs, degp, x, b_gcn, W1, b1, W2, b2, g1, be1, g2, be2)


def kernel(x, edge_index, W_gcn, b_gcn, W1, b1, W2, b2, g1, be1, g2, be2):
    ei5 = edge_index.astype(jnp.int32).reshape(
        2, NC * NS, NCHUNKS, CHUNK).transpose(1, 2, 0, 3)

    degp = _sc_degree(ei5)
    hs = _tc_scale(x, W_gcn, degp)
    sp = _sc_scatter(hs, ei5,
                     jnp.zeros((ROWS_PER_TILE, H), jnp.float32))
    return _tc_epilogue(sp, hs, degp, x, b_gcn, W1, b1, W2, b2,
                        g1, be1, g2, be2)


# speedup vs baseline: 1.4400x; 1.0303x over previous
import functools

import jax
import jax.numpy as jnp
from jax import lax
from jax.experimental import pallas as pl
from jax.experimental.pallas import tpu as pltpu
from jax.experimental.pallas import tpu_sc as plsc

N = 10000
E = 320000
D = 128
H = 128
FF = 256

NC = 2
NS = 16
CHUNK = 100
EDGES_PER_TILE = E // (NC * NS)
NCHUNKS = EDGES_PER_TILE // CHUNK
ONES_PAD = ((CHUNK + 15) // 16) * 16
NPAD = 10112
ROWS_PER_TILE = NPAD // NS
DEG_PAD = 10240
DEG_TILE = DEG_PAD // NS
RRING = 3
GDEPTH = RRING - 1
IRING = 8

ROW_BLOCK = 1024
GRID = 10

_MESH = plsc.VectorSubcoreMesh(
    core_axis_name="c", subcore_axis_name="s", num_cores=NC, num_subcores=NS)


@functools.partial(
    pl.kernel,
    mesh=_MESH,
    out_type=jax.ShapeDtypeStruct((NC, DEG_PAD), jnp.float32),
    scratch_types=[
        pltpu.VMEM((NCHUNKS, 2, CHUNK), jnp.int32),
        pltpu.VMEM((ONES_PAD,), jnp.float32),
        pltpu.VMEM((DEG_TILE,), jnp.float32),
        pltpu.SemaphoreType.DMA,
        pltpu.VMEM_SHARED((DEG_PAD,), jnp.float32),
    ],
)
def _sc_degree(ei_hbm, out_hbm, dstv, ones_v, zv, ssem, acc_sh):
    cid = lax.axis_index("c")
    sid = lax.axis_index("s")
    wid = cid * NS + sid

    def ones_body(i, c):
        ones_v[pl.ds(i * 16, 16)] = jnp.ones((16,), jnp.float32)
        return c
    lax.fori_loop(0, ONES_PAD // 16, ones_body, 0)

    def zeros_body(i, c):
        zv[pl.ds(i * 16, 16)] = jnp.zeros((16,), jnp.float32)
        return c
    lax.fori_loop(0, DEG_TILE // 16, zeros_body, 0)

    pltpu.sync_copy(ei_hbm.at[wid], dstv)
    pltpu.sync_copy(zv, acc_sh.at[pl.ds(sid * DEG_TILE, DEG_TILE)])
    plsc.subcore_barrier()

    def body(i, c):
        pltpu.async_copy(ones_v.at[pl.ds(0, CHUNK)], acc_sh.at[dstv.at[i, 1]], ssem, add=True)
        return c
    lax.fori_loop(0, NCHUNKS, body, 0)

    def drain(i, c):
        pltpu.make_async_copy(ones_v.at[pl.ds(0, CHUNK)], acc_sh.at[dstv.at[i, 1]], ssem).wait()
        return c
    lax.fori_loop(0, NCHUNKS, drain, 0)

    plsc.subcore_barrier()
    pltpu.sync_copy(acc_sh.at[pl.ds(sid * DEG_TILE, DEG_TILE)],
                    out_hbm.at[cid, pl.ds(sid * DEG_TILE, DEG_TILE)])


@functools.partial(
    pl.kernel,
    mesh=_MESH,
    out_type=jax.ShapeDtypeStruct((NC, NPAD, H), jnp.float32),
    scratch_types=[
        pltpu.VMEM((IRING, 2, CHUNK), jnp.int32),
        pltpu.VMEM((RRING, CHUNK, H), jnp.float32),
        pltpu.SemaphoreType.DMA,
        pltpu.SemaphoreType.DMA,
        pltpu.SemaphoreType.DMA,
        pltpu.VMEM_SHARED((NPAD, H), jnp.float32),
    ],
)
def _sc_scatter(hs_hbm, ei_hbm, z_hbm, out_hbm, idxv, rows_v,
                isem, gsem, ssem, acc_sh):
    cid = lax.axis_index("c")
    sid = lax.axis_index("s")
    wid = cid * NS + sid

    pltpu.sync_copy(z_hbm, acc_sh.at[pl.ds(sid * ROWS_PER_TILE, ROWS_PER_TILE)])

    def _load_idx(j, slot, sync):
        if sync:
            pltpu.sync_copy(ei_hbm.at[wid, j], idxv.at[slot])
        else:
            pltpu.async_copy(ei_hbm.at[wid, j], idxv.at[slot], isem)

    for j in range(GDEPTH):
        _load_idx(j, j, True)
        pltpu.async_copy(hs_hbm.at[idxv.at[j, 0]], rows_v.at[j], gsem)
    _load_idx(GDEPTH, GDEPTH, False)
    plsc.subcore_barrier()

    def body(i, c):
        b = lax.rem(i, RRING)
        q = i & (IRING - 1)
        pltpu.make_async_copy(hs_hbm.at[idxv.at[q, 0]],
                              rows_v.at[b], gsem).wait()
        pltpu.async_copy(rows_v.at[b], acc_sh.at[idxv.at[q, 1]],
                         ssem, add=True)

        @pl.when(i + GDEPTH < NCHUNKS)
        def _():
            m = (i + GDEPTH) & (IRING - 1)
            pltpu.make_async_copy(ei_hbm.at[wid, i + GDEPTH], idxv.at[m],
                                  isem).wait()
            r = lax.rem(i + GDEPTH, RRING)

            @pl.when(i >= 1)
            def _():
                pltpu.make_async_copy(
                    rows_v.at[r], acc_sh.at[idxv.at[(i - 1) & (IRING - 1), 1]],
                    ssem).wait()
            pltpu.async_copy(hs_hbm.at[idxv.at[m, 0]], rows_v.at[r], gsem)

        @pl.when(i + GDEPTH + 1 < NCHUNKS)
        def _():
            _load_idx(i + GDEPTH + 1, (i + GDEPTH + 1) & (IRING - 1), False)
        return c
    lax.fori_loop(0, NCHUNKS, body, 0)

    def sdrain(j, c):
        pltpu.make_async_copy(rows_v.at[lax.rem(j, RRING)],
                              acc_sh.at[idxv.at[j & (IRING - 1), 1]],
                              ssem).wait()
        return c
    lax.fori_loop(NCHUNKS - RRING, NCHUNKS, sdrain, 0)

    plsc.subcore_barrier()
    pltpu.sync_copy(acc_sh.at[pl.ds(sid * ROWS_PER_TILE, ROWS_PER_TILE)],
                    out_hbm.at[cid, pl.ds(sid * ROWS_PER_TILE, ROWS_PER_TILE)])


def _dinv_block(degp_blk):
    return lax.rsqrt(degp_blk[0] + degp_blk[1] + 1.0)[:, None]


def _tc1_body(x_ref, w_ref, degp_ref, hs_ref, dinv_ref):
    h = jnp.dot(x_ref[...], w_ref[...], preferred_element_type=jnp.float32)
    dinv = _dinv_block(degp_ref[...])
    hs_ref[...] = h * dinv
    dinv_ref[...] = dinv


def _tc_scale(x, W_gcn, degp):
    return pl.pallas_call(
        _tc1_body,
        grid=(GRID,),
        in_specs=[
            pl.BlockSpec((ROW_BLOCK, D), lambda i: (i, 0)),
            pl.BlockSpec((D, H), lambda i: (0, 0)),
            pl.BlockSpec((2, ROW_BLOCK), lambda i: (0, i)),
        ],
        out_specs=[
            pl.BlockSpec((ROW_BLOCK, H), lambda i: (i, 0)),
            pl.BlockSpec((ROW_BLOCK, 1), lambda i: (i, 0)),
        ],
        out_shape=[
            jax.ShapeDtypeStruct((N, H), jnp.float32),
            jax.ShapeDtypeStruct((N, 1), jnp.float32),
        ],
    )(x, W_gcn, degp)


def _ln(v, gamma, beta, eps=1e-5):
    mu = jnp.mean(v, axis=-1, keepdims=True)
    var = jnp.mean((v - mu) * (v - mu), axis=-1, keepdims=True)
    return (v - mu) * lax.rsqrt(var + eps) * gamma + beta


def _tc2_body(sp_ref, hs_ref, dinv_ref, x_ref, bg_ref, w1_ref, b1_ref, w2_ref,
              b2_ref, g1_ref, be1_ref, g2_ref, be2_ref, out_ref):
    s = sp_ref[0] + sp_ref[1]
    agg = dinv_ref[...] * (s + hs_ref[...]) + bg_ref[...]
    xr = x_ref[...] + agg
    xn = _ln(xr, g1_ref[...], be1_ref[...])
    t = jnp.maximum(
        jnp.dot(xn, w1_ref[...], preferred_element_type=jnp.float32) + b1_ref[...],
        0.0)
    ff = jnp.dot(t, w2_ref[...], preferred_element_type=jnp.float32) + b2_ref[...]
    out_ref[...] = _ln(xn + ff, g2_ref[...], be2_ref[...])


EPI_BLOCK = 1000


def _tc_epilogue(sp, hs, dinv2d, x, b_gcn, W1, b1, W2, b2, g1, be1, g2, be2):
    full = lambda shape: pl.BlockSpec(shape, lambda i: tuple(0 for _ in shape))
    return pl.pallas_call(
        _tc2_body,
        grid=(N // EPI_BLOCK,),
        in_specs=[
            pl.BlockSpec((NC, EPI_BLOCK, H), lambda i: (0, i, 0)),
            pl.BlockSpec((EPI_BLOCK, H), lambda i: (i, 0)),
            pl.BlockSpec((EPI_BLOCK, 1), lambda i: (i, 0)),
            pl.BlockSpec((EPI_BLOCK, D), lambda i: (i, 0)),
            full((H,)),
            full((H, FF)),
            full((FF,)),
            full((FF, H)),
            full((H,)),
            full((H,)),
            full((H,)),
            full((H,)),
            full((H,)),
        ],
        out_specs=pl.BlockSpec((EPI_BLOCK, H), lambda i: (i, 0)),
        out_shape=jax.ShapeDtypeStruct((N, H), jnp.float32),
    )(sp, hs, dinv2d, x, b_gcn, W1, b1, W2, b2, g1, be1, g2, be2)


def kernel(x, edge_index, W_gcn, b_gcn, W1, b1, W2, b2, g1, be1, g2, be2):
    ei5 = edge_index.astype(jnp.int32).reshape(
        2, NC * NS, NCHUNKS, CHUNK).transpose(1, 2, 0, 3)

    degp = _sc_degree(ei5)
    hs, dinv2d = _tc_scale(x, W_gcn, degp)
    sp = _sc_scatter(hs, ei5,
                     jnp.zeros((ROWS_PER_TILE, H), jnp.float32))
    return _tc_epilogue(sp, hs, dinv2d, x, b_gcn, W1, b1, W2, b2,
                        g1, be1, g2, be2)
